# baseline (speedup 1.0000x reference)
import jax
import jax.numpy as jnp
from jax.experimental import pallas as pl

_R = 512


def _emb_kernel(x_ref, w_ref, o_ref):
    xb = x_ref[...]
    nv = xb.shape[1]
    m = jnp.max(xb, axis=1, keepdims=True)
    iota = jax.lax.broadcasted_iota(jnp.int32, xb.shape, 1)
    idx = jnp.min(jnp.where(xb == m, iota, nv), axis=1, keepdims=True)
    onehot = (iota == idx).astype(jnp.float32)
    o_ref[...] = jnp.dot(onehot, w_ref[...],
                         preferred_element_type=jnp.float32)


def kernel(x, W):
    B, S, NV = x.shape
    E = W.shape[1]
    x2 = x.reshape(B * S, NV)
    out = pl.pallas_call(
        _emb_kernel,
        grid=(B * S // _R,),
        in_specs=[
            pl.BlockSpec((_R, NV), lambda i: (i, 0)),
            pl.BlockSpec((NV, E), lambda i: (0, 0)),
        ],
        out_specs=pl.BlockSpec((_R, E), lambda i: (i, 0)),
        out_shape=jax.ShapeDtypeStruct((B * S, E), jnp.float32),
    )(x2, W)
    return out.reshape(B, S, E)
